# baseline (device time: 98732 ns/iter reference)
import jax
import jax.numpy as jnp
from jax import lax
from jax.experimental import pallas as pl
from jax.experimental.pallas import tpu as pltpu

N_DEV = 4
SUB = 4


def kernel(t, W):
    m_per, k = t.shape
    _, n = W.shape
    blk = m_per // N_DEV
    half = k // 2
    rows = blk // SUB

    SRC = [3, 0, 1, 2, 3, 0]
    DST = [0, 1, 2, 3, 0, 1]
    N_HOPS = 6

    def body(t_ref, w_ref, out_ref, cw_ref, ccw_ref,
             cw_send, cw_recv, ccw_send, ccw_recv):
        p = lax.axis_index("i")
        left = lax.rem(p + N_DEV - 1, N_DEV)
        right = lax.rem(p + 1, N_DEV)

        barrier_sem = pltpu.get_barrier_semaphore()
        for nbr in (left, right):
            pl.semaphore_signal(
                barrier_sem, inc=1,
                device_id=(nbr,), device_id_type=pl.DeviceIdType.MESH,
            )
        pl.semaphore_wait(barrier_sem, 2)

        bf16 = jnp.bfloat16
        f32 = jnp.float32

        def t_cw(c, sub):
            return t_ref[pl.ds(lax.rem(c, N_DEV) * blk + sub * rows, rows),
                         0:half]

        def t_ccw(c, sub):
            return t_ref[pl.ds(lax.rem(c, N_DEV) * blk + sub * rows, rows),
                         half:k]

        dsc = {}
        for h in range(N_HOPS):
            for sub in range(SUB):
                r0 = sub * rows
                i = h * SUB + sub
                dsc["cw", h, sub] = pltpu.make_async_remote_copy(
                    src_ref=cw_ref.at[SRC[h], pl.ds(r0, rows), :],
                    dst_ref=cw_ref.at[DST[h], pl.ds(r0, rows), :],
                    send_sem=cw_send.at[i],
                    recv_sem=cw_recv.at[i],
                    device_id=(right,),
                    device_id_type=pl.DeviceIdType.MESH,
                )
                dsc["ccw", h, sub] = pltpu.make_async_remote_copy(
                    src_ref=ccw_ref.at[SRC[h], pl.ds(r0, rows), :],
                    dst_ref=ccw_ref.at[DST[h], pl.ds(r0, rows), :],
                    send_sem=ccw_send.at[i],
                    recv_sem=ccw_recv.at[i],
                    device_id=(left,),
                    device_id_type=pl.DeviceIdType.MESH,
                )

        def start(h, sub):
            dsc["cw", h, sub].start()
            dsc["ccw", h, sub].start()

        def wait_recv(h, sub):
            dsc["cw", h, sub].wait_recv()
            dsc["ccw", h, sub].wait_recv()

        for sub in range(SUB):
            r0 = sub * rows
            cw_ref[3, pl.ds(r0, rows), :] = t_cw(p, sub).astype(bf16)
            ccw_ref[3, pl.ds(r0, rows), :] = t_ccw(p + 2, sub).astype(bf16)
            start(0, sub)

        cw_acc = [p + 3, p + 2]
        ccw_acc = [p + 3, p]
        for h in (1, 2):
            slot = DST[h - 1]
            for sub in range(SUB):
                r0 = sub * rows
                wait_recv(h - 1, sub)
                cw_ref[slot, pl.ds(r0, rows), :] = (
                    cw_ref[slot, pl.ds(r0, rows), :].astype(f32)
                    + t_cw(cw_acc[h - 1], sub)
                ).astype(bf16)
                ccw_ref[slot, pl.ds(r0, rows), :] = (
                    ccw_ref[slot, pl.ds(r0, rows), :].astype(f32)
                    + t_ccw(ccw_acc[h - 1], sub)
                ).astype(bf16)
                start(h, sub)

        q = lax.rem(p + 1, N_DEV)
        w_bf = w_ref[:, :].astype(bf16)
        for sub in range(SUB):
            r0 = sub * rows
            wait_recv(2, sub)
            s_l = cw_ref[2, pl.ds(r0, rows), :].astype(f32) + t_cw(p + 1, sub)
            s_r = ccw_ref[2, pl.ds(r0, rows), :].astype(f32) + t_ccw(p + 1, sub)
            s_sub = jnp.concatenate([s_l, s_r], axis=1).astype(bf16)
            o = jnp.dot(s_sub, w_bf, preferred_element_type=f32)
            cw_ref[2, pl.ds(r0, rows), :] = o[:, 0:half].astype(bf16)
            ccw_ref[2, pl.ds(r0, rows), :] = o[:, half:k].astype(bf16)
            start(3, sub)
            out_ref[pl.ds(q * blk + r0, rows), :] = o

        cw_store = [p, p + 3, p + 2]
        ccw_store = [p + 2, p + 3, p]
        for h in (4, 5):
            slot = DST[h - 1]
            g = h - 4
            for sub in range(SUB):
                r0 = sub * rows
                wait_recv(h - 1, sub)
                start(h, sub)
                out_ref[pl.ds(lax.rem(cw_store[g], N_DEV) * blk + r0, rows),
                        0:half] = cw_ref[slot, pl.ds(r0, rows), :].astype(f32)
                out_ref[pl.ds(lax.rem(ccw_store[g], N_DEV) * blk + r0, rows),
                        half:k] = ccw_ref[slot, pl.ds(r0, rows), :].astype(f32)

        for sub in range(SUB):
            r0 = sub * rows
            wait_recv(5, sub)
            out_ref[pl.ds(lax.rem(cw_store[2], N_DEV) * blk + r0, rows),
                    0:half] = cw_ref[1, pl.ds(r0, rows), :].astype(f32)
            out_ref[pl.ds(lax.rem(ccw_store[2], N_DEV) * blk + r0, rows),
                    half:k] = ccw_ref[1, pl.ds(r0, rows), :].astype(f32)

        for h in range(N_HOPS):
            for sub in range(SUB):
                dsc["cw", h, sub].wait_send()
                dsc["ccw", h, sub].wait_send()

    n_sems = N_HOPS * SUB
    return pl.pallas_call(
        body,
        out_shape=jax.ShapeDtypeStruct((m_per, n), jnp.float32),
        in_specs=[
            pl.BlockSpec(memory_space=pltpu.VMEM),
            pl.BlockSpec(memory_space=pltpu.VMEM),
        ],
        out_specs=pl.BlockSpec(memory_space=pltpu.VMEM),
        scratch_shapes=[
            pltpu.VMEM((4, blk, half), jnp.bfloat16),
            pltpu.VMEM((4, blk, half), jnp.bfloat16),
            pltpu.SemaphoreType.DMA((n_sems,)),
            pltpu.SemaphoreType.DMA((n_sems,)),
            pltpu.SemaphoreType.DMA((n_sems,)),
            pltpu.SemaphoreType.DMA((n_sems,)),
        ],
        compiler_params=pltpu.CompilerParams(
            collective_id=0,
            vmem_limit_bytes=100 * 1024 * 1024,
        ),
    )(t, W)


# device time: 89136 ns/iter; 1.1077x vs baseline; 1.1077x over previous
import jax
import jax.numpy as jnp
from jax import lax
from jax.experimental import pallas as pl
from jax.experimental.pallas import tpu as pltpu

N_DEV = 4
SUB = 2


def kernel(t, W):
    m_per, k = t.shape
    _, n = W.shape
    blk = m_per // N_DEV
    half = k // 2
    rows = blk // SUB

    SRC = [3, 0, 1, 2, 3, 0]
    DST = [0, 1, 2, 3, 0, 1]
    N_HOPS = 6

    def body(t_ref, w_ref, out_ref, cw_ref, ccw_ref, tcw_v, tccw_v, w_v, ob_v,
             cw_send, cw_recv, ccw_send, ccw_recv, t_sems, w_sem, out_sems):
        p = lax.axis_index("i")
        left = lax.rem(p + N_DEV - 1, N_DEV)
        right = lax.rem(p + 1, N_DEV)

        bf16 = jnp.bfloat16
        f32 = jnp.float32

        def blk_rows(c):
            return pl.ds(lax.rem(c, N_DEV) * blk, blk)

        cw_blocks = [p, p + 3, p + 2, p + 1]
        ccw_blocks = [p + 2, p + 3, p, p + 1]
        t_cps = []
        for i in range(4):
            cp_cw = pltpu.make_async_copy(
                t_ref.at[blk_rows(cw_blocks[i]), pl.ds(0, half)],
                tcw_v.at[i], t_sems.at[2 * i],
            )
            cp_ccw = pltpu.make_async_copy(
                t_ref.at[blk_rows(ccw_blocks[i]), pl.ds(half, half)],
                tccw_v.at[i], t_sems.at[2 * i + 1],
            )
            cp_cw.start()
            cp_ccw.start()
            t_cps.append((cp_cw, cp_ccw))
        w_cp = pltpu.make_async_copy(w_ref, w_v, w_sem)
        w_cp.start()

        barrier_sem = pltpu.get_barrier_semaphore()
        for nbr in (left, right):
            pl.semaphore_signal(
                barrier_sem, inc=1,
                device_id=(nbr,), device_id_type=pl.DeviceIdType.MESH,
            )
        pl.semaphore_wait(barrier_sem, 2)

        dsc = {}
        for h in range(N_HOPS):
            for sub in range(SUB):
                r0 = sub * rows
                i = h * SUB + sub
                dsc["cw", h, sub] = pltpu.make_async_remote_copy(
                    src_ref=cw_ref.at[SRC[h], pl.ds(r0, rows), :],
                    dst_ref=cw_ref.at[DST[h], pl.ds(r0, rows), :],
                    send_sem=cw_send.at[i],
                    recv_sem=cw_recv.at[i],
                    device_id=(right,),
                    device_id_type=pl.DeviceIdType.MESH,
                )
                dsc["ccw", h, sub] = pltpu.make_async_remote_copy(
                    src_ref=ccw_ref.at[SRC[h], pl.ds(r0, rows), :],
                    dst_ref=ccw_ref.at[DST[h], pl.ds(r0, rows), :],
                    send_sem=ccw_send.at[i],
                    recv_sem=ccw_recv.at[i],
                    device_id=(left,),
                    device_id_type=pl.DeviceIdType.MESH,
                )

        def start(h, sub):
            dsc["cw", h, sub].start()
            dsc["ccw", h, sub].start()

        def wait_recv(h, sub):
            dsc["cw", h, sub].wait_recv()
            dsc["ccw", h, sub].wait_recv()

        t_cps[0][0].wait()
        t_cps[0][1].wait()
        for sub in range(SUB):
            r0 = sub * rows
            cw_ref[3, pl.ds(r0, rows), :] = (
                tcw_v[0, pl.ds(r0, rows), :].astype(bf16))
            ccw_ref[3, pl.ds(r0, rows), :] = (
                tccw_v[0, pl.ds(r0, rows), :].astype(bf16))
            start(0, sub)

        for h in (1, 2):
            slot = DST[h - 1]
            t_cps[h][0].wait()
            t_cps[h][1].wait()
            for sub in range(SUB):
                r0 = sub * rows
                wait_recv(h - 1, sub)
                cw_ref[slot, pl.ds(r0, rows), :] = (
                    cw_ref[slot, pl.ds(r0, rows), :].astype(f32)
                    + tcw_v[h, pl.ds(r0, rows), :]
                ).astype(bf16)
                ccw_ref[slot, pl.ds(r0, rows), :] = (
                    ccw_ref[slot, pl.ds(r0, rows), :].astype(f32)
                    + tccw_v[h, pl.ds(r0, rows), :]
                ).astype(bf16)
                start(h, sub)

        q = lax.rem(p + 1, N_DEV)
        t_cps[3][0].wait()
        t_cps[3][1].wait()
        w_cp.wait()
        w_bf = w_v[:, :].astype(bf16)
        for sub in range(SUB):
            r0 = sub * rows
            wait_recv(2, sub)
            s_l = cw_ref[2, pl.ds(r0, rows), :].astype(f32) + tcw_v[3, pl.ds(r0, rows), :]
            s_r = ccw_ref[2, pl.ds(r0, rows), :].astype(f32) + tccw_v[3, pl.ds(r0, rows), :]
            s_sub = jnp.concatenate([s_l, s_r], axis=1).astype(bf16)
            o = jnp.dot(s_sub, w_bf, preferred_element_type=f32)
            cw_ref[2, pl.ds(r0, rows), :] = o[:, 0:half].astype(bf16)
            ccw_ref[2, pl.ds(r0, rows), :] = o[:, half:k].astype(bf16)
            start(3, sub)
            ob_v[0, pl.ds(r0, rows), :] = o

        out_cps = []
        cp = pltpu.make_async_copy(
            ob_v.at[0], out_ref.at[pl.ds(q * blk, blk), :], out_sems.at[0])
        cp.start()
        out_cps.append(cp)

        cw_store = [p, p + 3, p + 2]
        ccw_store = [p + 2, p + 3, p]

        def flush_ag(g):
            sidx = 1 + 2 * g
            cp_l = pltpu.make_async_copy(
                ob_v.at[1 + g, :, pl.ds(0, half)],
                out_ref.at[blk_rows(cw_store[g]), pl.ds(0, half)],
                out_sems.at[sidx],
            )
            cp_r = pltpu.make_async_copy(
                ob_v.at[1 + g, :, pl.ds(half, half)],
                out_ref.at[blk_rows(ccw_store[g]), pl.ds(half, half)],
                out_sems.at[sidx + 1],
            )
            cp_l.start()
            cp_r.start()
            out_cps.append(cp_l)
            out_cps.append(cp_r)

        for h in (4, 5):
            slot = DST[h - 1]
            g = h - 4
            for sub in range(SUB):
                r0 = sub * rows
                wait_recv(h - 1, sub)
                start(h, sub)
                ob_v[1 + g, pl.ds(r0, rows), 0:half] = (
                    cw_ref[slot, pl.ds(r0, rows), :].astype(f32))
                ob_v[1 + g, pl.ds(r0, rows), half:k] = (
                    ccw_ref[slot, pl.ds(r0, rows), :].astype(f32))
            flush_ag(g)

        for sub in range(SUB):
            r0 = sub * rows
            wait_recv(5, sub)
            ob_v[3, pl.ds(r0, rows), 0:half] = (
                cw_ref[1, pl.ds(r0, rows), :].astype(f32))
            ob_v[3, pl.ds(r0, rows), half:k] = (
                ccw_ref[1, pl.ds(r0, rows), :].astype(f32))
        flush_ag(2)

        for cp in out_cps:
            cp.wait()
        for h in range(N_HOPS):
            for sub in range(SUB):
                dsc["cw", h, sub].wait_send()
                dsc["ccw", h, sub].wait_send()

    n_sems = N_HOPS * SUB
    return pl.pallas_call(
        body,
        out_shape=jax.ShapeDtypeStruct((m_per, n), jnp.float32),
        in_specs=[
            pl.BlockSpec(memory_space=pl.ANY),
            pl.BlockSpec(memory_space=pl.ANY),
        ],
        out_specs=pl.BlockSpec(memory_space=pl.ANY),
        scratch_shapes=[
            pltpu.VMEM((4, blk, half), jnp.bfloat16),
            pltpu.VMEM((4, blk, half), jnp.bfloat16),
            pltpu.VMEM((4, blk, half), jnp.float32),
            pltpu.VMEM((4, blk, half), jnp.float32),
            pltpu.VMEM((k, n), jnp.float32),
            pltpu.VMEM((4, blk, n), jnp.float32),
            pltpu.SemaphoreType.DMA((n_sems,)),
            pltpu.SemaphoreType.DMA((n_sems,)),
            pltpu.SemaphoreType.DMA((n_sems,)),
            pltpu.SemaphoreType.DMA((n_sems,)),
            pltpu.SemaphoreType.DMA((8,)),
            pltpu.SemaphoreType.DMA(()),
            pltpu.SemaphoreType.DMA((8,)),
        ],
        compiler_params=pltpu.CompilerParams(
            collective_id=0,
            vmem_limit_bytes=100 * 1024 * 1024,
        ),
    )(t, W)


# device time: 88561 ns/iter; 1.1148x vs baseline; 1.0065x over previous
import jax
import jax.numpy as jnp
from jax import lax
from jax.experimental import pallas as pl
from jax.experimental.pallas import tpu as pltpu

N_DEV = 4
SUB = 2


def kernel(t, W):
    m_per, k = t.shape
    _, n = W.shape
    blk = m_per // N_DEV
    half = k // 2
    rows = blk // SUB

    SRC = [3, 0, 1, 2, 3, 0]
    DST = [0, 1, 2, 3, 0, 1]
    N_HOPS = 6

    def body(t_ref, w_ref, out_ref, cw_ref, ccw_ref, tcw_v, tccw_v, w_v, ob_v,
             cw_send, cw_recv, ccw_send, ccw_recv, t_sems, w_sem, out_sems):
        p = lax.axis_index("i")
        left = lax.rem(p + N_DEV - 1, N_DEV)
        right = lax.rem(p + 1, N_DEV)

        bf16 = jnp.bfloat16
        f32 = jnp.float32

        def blk_rows(c):
            return pl.ds(lax.rem(c, N_DEV) * blk, blk)

        cw_blocks = [p, p + 3, p + 2, p + 1]
        ccw_blocks = [p + 2, p + 3, p, p + 1]
        t_cps = []
        for i in range(4):
            cp_cw = pltpu.make_async_copy(
                t_ref.at[blk_rows(cw_blocks[i]), pl.ds(0, half)],
                tcw_v.at[i], t_sems.at[2 * i],
            )
            cp_ccw = pltpu.make_async_copy(
                t_ref.at[blk_rows(ccw_blocks[i]), pl.ds(half, half)],
                tccw_v.at[i], t_sems.at[2 * i + 1],
            )
            cp_cw.start()
            cp_ccw.start()
            t_cps.append((cp_cw, cp_ccw))
        w_cp = pltpu.make_async_copy(w_ref, w_v, w_sem)
        w_cp.start()

        barrier_sem = pltpu.get_barrier_semaphore()
        for nbr in (left, right):
            pl.semaphore_signal(
                barrier_sem, inc=1,
                device_id=(nbr,), device_id_type=pl.DeviceIdType.MESH,
            )
        pl.semaphore_wait(barrier_sem, 2)

        dsc = {}
        for h in range(N_HOPS):
            for sub in range(SUB):
                r0 = sub * rows
                i = h * SUB + sub
                dsc["cw", h, sub] = pltpu.make_async_remote_copy(
                    src_ref=cw_ref.at[SRC[h], pl.ds(r0, rows), :],
                    dst_ref=cw_ref.at[DST[h], pl.ds(r0, rows), :],
                    send_sem=cw_send.at[i],
                    recv_sem=cw_recv.at[i],
                    device_id=(right,),
                    device_id_type=pl.DeviceIdType.MESH,
                )
                dsc["ccw", h, sub] = pltpu.make_async_remote_copy(
                    src_ref=ccw_ref.at[SRC[h], pl.ds(r0, rows), :],
                    dst_ref=ccw_ref.at[DST[h], pl.ds(r0, rows), :],
                    send_sem=ccw_send.at[i],
                    recv_sem=ccw_recv.at[i],
                    device_id=(left,),
                    device_id_type=pl.DeviceIdType.MESH,
                )

        def start(h, sub):
            dsc["cw", h, sub].start()
            dsc["ccw", h, sub].start()

        def wait_recv(h, sub):
            dsc["cw", h, sub].wait_recv()
            dsc["ccw", h, sub].wait_recv()

        t_cps[0][0].wait()
        t_cps[0][1].wait()
        for sub in range(SUB):
            r0 = sub * rows
            cw_ref[3, pl.ds(r0, rows), :] = (
                tcw_v[0, pl.ds(r0, rows), :].astype(bf16))
            ccw_ref[3, pl.ds(r0, rows), :] = (
                tccw_v[0, pl.ds(r0, rows), :].astype(bf16))
            start(0, sub)

        for h in (1, 2):
            slot = DST[h - 1]
            t_cps[h][0].wait()
            t_cps[h][1].wait()
            for sub in range(SUB):
                r0 = sub * rows
                wait_recv(h - 1, sub)
                cw_ref[slot, pl.ds(r0, rows), :] = (
                    cw_ref[slot, pl.ds(r0, rows), :].astype(f32)
                    + tcw_v[h, pl.ds(r0, rows), :]
                ).astype(bf16)
                ccw_ref[slot, pl.ds(r0, rows), :] = (
                    ccw_ref[slot, pl.ds(r0, rows), :].astype(f32)
                    + tccw_v[h, pl.ds(r0, rows), :]
                ).astype(bf16)
                start(h, sub)

        q = lax.rem(p + 1, N_DEV)
        t_cps[3][0].wait()
        t_cps[3][1].wait()
        w_cp.wait()
        w_bf = w_v[:, :].astype(bf16)
        for sub in range(SUB):
            r0 = sub * rows
            wait_recv(2, sub)
            s_l = cw_ref[2, pl.ds(r0, rows), :].astype(f32) + tcw_v[3, pl.ds(r0, rows), :]
            s_r = ccw_ref[2, pl.ds(r0, rows), :].astype(f32) + tccw_v[3, pl.ds(r0, rows), :]
            s_sub = jnp.concatenate([s_l, s_r], axis=1).astype(bf16)
            o = jnp.dot(s_sub, w_bf, preferred_element_type=f32)
            cw_ref[2, pl.ds(r0, rows), :] = o[:, 0:half].astype(bf16)
            ccw_ref[2, pl.ds(r0, rows), :] = o[:, half:k].astype(bf16)
            start(3, sub)
            ob_v[0, pl.ds(r0, rows), :] = o

        out_cps = []

        def flush_own(sub):
            r0 = sub * rows
            cp = pltpu.make_async_copy(
                ob_v.at[0, pl.ds(r0, rows), :],
                out_ref.at[pl.ds(q * blk + r0, rows), :],
                out_sems.at[sub],
            )
            cp.start()
            out_cps.append(cp)

        for sub in range(SUB):
            flush_own(sub)

        cw_store = [p, p + 3, p + 2]
        ccw_store = [p + 2, p + 3, p]

        def store_flush_ag(g, slot, sub):
            r0 = sub * rows
            ob_v[1 + g, pl.ds(r0, rows), 0:half] = (
                cw_ref[slot, pl.ds(r0, rows), :].astype(f32))
            ob_v[1 + g, pl.ds(r0, rows), half:k] = (
                ccw_ref[slot, pl.ds(r0, rows), :].astype(f32))
            sidx = SUB + 4 * g + 2 * sub
            cp_l = pltpu.make_async_copy(
                ob_v.at[1 + g, pl.ds(r0, rows), pl.ds(0, half)],
                out_ref.at[pl.ds(lax.rem(cw_store[g], N_DEV) * blk + r0, rows),
                           pl.ds(0, half)],
                out_sems.at[sidx],
            )
            cp_r = pltpu.make_async_copy(
                ob_v.at[1 + g, pl.ds(r0, rows), pl.ds(half, half)],
                out_ref.at[pl.ds(lax.rem(ccw_store[g], N_DEV) * blk + r0, rows),
                           pl.ds(half, half)],
                out_sems.at[sidx + 1],
            )
            cp_l.start()
            cp_r.start()
            out_cps.append(cp_l)
            out_cps.append(cp_r)

        for h in (4, 5):
            slot = DST[h - 1]
            g = h - 4
            for sub in range(SUB):
                wait_recv(h - 1, sub)
                start(h, sub)
                store_flush_ag(g, slot, sub)

        for sub in range(SUB):
            wait_recv(5, sub)
            store_flush_ag(2, 1, sub)

        for cp in out_cps:
            cp.wait()
        for h in range(N_HOPS):
            for sub in range(SUB):
                dsc["cw", h, sub].wait_send()
                dsc["ccw", h, sub].wait_send()

    n_sems = N_HOPS * SUB
    return pl.pallas_call(
        body,
        out_shape=jax.ShapeDtypeStruct((m_per, n), jnp.float32),
        in_specs=[
            pl.BlockSpec(memory_space=pl.ANY),
            pl.BlockSpec(memory_space=pl.ANY),
        ],
        out_specs=pl.BlockSpec(memory_space=pl.ANY),
        scratch_shapes=[
            pltpu.VMEM((4, blk, half), jnp.bfloat16),
            pltpu.VMEM((4, blk, half), jnp.bfloat16),
            pltpu.VMEM((4, blk, half), jnp.float32),
            pltpu.VMEM((4, blk, half), jnp.float32),
            pltpu.VMEM((k, n), jnp.float32),
            pltpu.VMEM((4, blk, n), jnp.float32),
            pltpu.SemaphoreType.DMA((n_sems,)),
            pltpu.SemaphoreType.DMA((n_sems,)),
            pltpu.SemaphoreType.DMA((n_sems,)),
            pltpu.SemaphoreType.DMA((n_sems,)),
            pltpu.SemaphoreType.DMA((8,)),
            pltpu.SemaphoreType.DMA(()),
            pltpu.SemaphoreType.DMA((16,)),
        ],
        compiler_params=pltpu.CompilerParams(
            collective_id=0,
            vmem_limit_bytes=100 * 1024 * 1024,
        ),
    )(t, W)
